# baseline (speedup 1.0000x reference)
import functools

import jax
import jax.numpy as jnp
from jax import lax
from jax.experimental import pallas as pl
from jax.experimental.pallas import tpu as pltpu
from jax.experimental.pallas import tpu_sc as plsc

_K = 1024
_D = 32
_N = 4096
_BT = 1024
_H = _N // 2
_GRID = _H // _BT
_NC, _NS = 2, 16
_NW = _NC * _NS
_BPW = _H // _NW


def _dist_body(xt_ref, w_ref, xsq_ref, wsq_ref, idx_ref, loss_ref, acc_ref):
    xt = xt_ref[...]
    w2 = w_ref[...]
    xsq = xsq_ref[...]
    wsq = wsq_ref[...]
    mm2 = lax.dot_general(xt, w2, (((0,), (1,)), ((), ())),
                          preferred_element_type=jnp.float32)
    d = (xsq + wsq) + mm2
    m = jnp.min(d, axis=1, keepdims=True)
    iota = lax.broadcasted_iota(jnp.int32, (_BT, _K), 1)
    idx = jnp.min(jnp.where(d == m, iota, _K), axis=1).astype(jnp.int32)
    idx_ref[...] = idx.reshape(1, 1, _BT)
    part = jnp.sum(m)

    i = pl.program_id(0)

    @pl.when(i == 0)
    def _init():
        acc_ref[0] = 0.0

    acc_ref[0] += part

    @pl.when(i == _GRID - 1)
    def _fini():
        loss_ref[0, 0] = acc_ref[0]


def _distances_half(x_cn, w2, xsq, wsq, half):
    boff = half * _GRID
    return pl.pallas_call(
        _dist_body,
        grid=(_GRID,),
        in_specs=[
            pl.BlockSpec((_D, _BT), lambda i: (boff + i, 0)),
            pl.BlockSpec((_K, _D), lambda i: (0, 0)),
            pl.BlockSpec((_BT, 1), lambda i: (boff + i, 0)),
            pl.BlockSpec((1, _K), lambda i: (0, 0)),
        ],
        out_specs=[
            pl.BlockSpec((1, 1, _BT), lambda i: (i, 0, 0)),
            pl.BlockSpec(memory_space=pltpu.SMEM),
        ],
        out_shape=[
            jax.ShapeDtypeStruct((_GRID, 1, _BT), jnp.int32),
            jax.ShapeDtypeStruct((1, 1), jnp.float32),
        ],
        scratch_shapes=[pltpu.SMEM((1,), jnp.float32)],
    )(x_cn, w2, xsq, wsq)


def _sc_gather(weight, idx_flat):
    mesh = plsc.VectorSubcoreMesh(core_axis_name="c", subcore_axis_name="s")

    @functools.partial(
        pl.kernel,
        mesh=mesh,
        out_type=jax.ShapeDtypeStruct((_H, _D), jnp.float32),
        scratch_types=[
            pltpu.VMEM((_BPW,), jnp.int32),
            pltpu.VMEM((_BPW, _D), jnp.float32),
            pltpu.SemaphoreType.DMA,
        ],
        compiler_params=pltpu.CompilerParams(use_tc_tiling_on_sc=False),
    )
    def gather_k(w_hbm, idx_hbm, out_hbm, idx_v, rows_v, sem):
        wid = lax.axis_index("s") * _NC + lax.axis_index("c")
        base = wid * _BPW
        pltpu.sync_copy(idx_hbm.at[pl.ds(base, _BPW)], idx_v)
        pltpu.async_copy(w_hbm.at[idx_v], rows_v, sem).wait()
        pltpu.sync_copy(rows_v, out_hbm.at[pl.ds(base, _BPW)])

    return gather_k(weight, idx_flat)


def kernel(inputs, weight):
    x_cn = inputs.reshape(4 * _D, 32 * 32)
    xsq = jnp.sum(jnp.transpose(inputs, (0, 2, 3, 1)).reshape(-1, _D) ** 2,
                  axis=1, keepdims=True)
    wsq = jnp.sum(weight ** 2, axis=1).reshape(1, _K)
    w2 = weight * (-2.0)
    idx_a, loss_a = _distances_half(x_cn, w2, xsq, wsq, 0)
    q_a = _sc_gather(weight, idx_a.reshape(_H))
    idx_b, loss_b = _distances_half(x_cn, w2, xsq, wsq, 1)
    q_b = _sc_gather(weight, idx_b.reshape(_H))
    loss = (loss_a[0, 0] + loss_b[0, 0]) * (1.25 / (_N * _D))
    q = jnp.concatenate([q_a, q_b], axis=0)
    quantized_st = jnp.transpose(q.reshape(4, 32, 32, _D), (0, 3, 1, 2))
    idx3 = jnp.concatenate([idx_a, idx_b], axis=0).reshape(4, 32, 32)
    return quantized_st, loss, idx3


# speedup vs baseline: 1.8467x; 1.8467x over previous
import jax
import jax.numpy as jnp
from jax import lax
from jax.experimental import pallas as pl
from jax.experimental.pallas import tpu as pltpu

_K = 1024
_D = 32
_N = 4096
_BT = 1024
_GRID = _N // _BT


def _vq_body(xt_ref, w_ref, xsq_ref, wsq_ref, idx_ref, loss_ref, qt_ref,
             acc_ref):
    xt = xt_ref[...]
    w = w_ref[...]
    xsq = xsq_ref[...]
    wsq = wsq_ref[...]
    mm = lax.dot_general(xt, w, (((0,), (1,)), ((), ())),
                         preferred_element_type=jnp.float32)
    d = (xsq + wsq) + (-2.0) * mm
    m = jnp.min(d, axis=1, keepdims=True)
    iota = lax.broadcasted_iota(jnp.int32, (_BT, _K), 1)
    idx = jnp.min(jnp.where(d == m, iota, _K), axis=1).astype(jnp.int32)
    idx_ref[...] = idx.reshape(1, 1, _BT)
    onehot = jnp.where(iota == idx.reshape(_BT, 1), 1.0, 0.0)
    qt = lax.dot_general(w, onehot, (((0,), (1,)), ((), ())),
                         preferred_element_type=jnp.float32)
    qt_ref[...] = qt.reshape(1, _D, _BT)
    part = jnp.sum(m)

    i = pl.program_id(0)

    @pl.when(i == 0)
    def _init():
        acc_ref[0] = 0.0

    acc_ref[0] += part

    @pl.when(i == _GRID - 1)
    def _fini():
        loss_ref[0, 0] = acc_ref[0] * (1.25 / (_N * _D))


def _vq(x_cn, weight, xsq, wsq):
    return pl.pallas_call(
        _vq_body,
        grid=(_GRID,),
        in_specs=[
            pl.BlockSpec((_D, _BT), lambda i: (i, 0)),
            pl.BlockSpec((_K, _D), lambda i: (0, 0)),
            pl.BlockSpec((_BT, 1), lambda i: (i, 0)),
            pl.BlockSpec((1, _K), lambda i: (0, 0)),
        ],
        out_specs=[
            pl.BlockSpec((1, 1, _BT), lambda i: (i, 0, 0)),
            pl.BlockSpec(memory_space=pltpu.SMEM),
            pl.BlockSpec((1, _D, _BT), lambda i: (i, 0, 0)),
        ],
        out_shape=[
            jax.ShapeDtypeStruct((_GRID, 1, _BT), jnp.int32),
            jax.ShapeDtypeStruct((1, 1), jnp.float32),
            jax.ShapeDtypeStruct((_GRID, _D, _BT), jnp.float32),
        ],
        scratch_shapes=[pltpu.SMEM((1,), jnp.float32)],
    )(x_cn, weight, xsq, wsq)


def kernel(inputs, weight):
    x_cn = inputs.reshape(4 * _D, 32 * 32)
    xsq = jnp.sum(jnp.transpose(inputs, (0, 2, 3, 1)).reshape(-1, _D) ** 2,
                  axis=1, keepdims=True)
    wsq = jnp.sum(weight ** 2, axis=1).reshape(1, _K)
    idx3, loss, qt = _vq(x_cn, weight, xsq, wsq)
    quantized_st = qt.reshape(4, _D, 32, 32)
    return quantized_st, loss[0, 0], idx3.reshape(4, 32, 32)


# speedup vs baseline: 1.9270x; 1.0435x over previous
import jax
import jax.numpy as jnp
from jax import lax
from jax.experimental import pallas as pl
from jax.experimental.pallas import tpu as pltpu

_K = 1024
_D = 32
_N = 4096
_BT = 1024
_GRID = _N // _BT


def _vq_body(xt_ref, w_ref, xsq_ref, wsq_ref, idx_ref, loss_ref, qt_ref,
             acc_ref):
    xt = xt_ref[...]
    w2 = w_ref[...] * (-2.0)
    xsq = xsq_ref[...]
    wsq = wsq_ref[...]
    mm2 = lax.dot_general(xt, w2, (((0,), (1,)), ((), ())),
                          preferred_element_type=jnp.float32)
    d = (xsq + wsq) + mm2
    m = jnp.min(d, axis=1, keepdims=True)
    iota = lax.broadcasted_iota(jnp.int32, (1, _K), 1).astype(jnp.float32)
    idxf = jnp.min(jnp.where(d == m, iota, jnp.float32(_K)), axis=1,
                   keepdims=True)
    idx_ref[...] = idxf.astype(jnp.int32).reshape(1, 1, _BT)
    onehot = jnp.where(iota == idxf, 1.0, 0.0)
    qt = lax.dot_general(w2, onehot, (((0,), (1,)), ((), ())),
                         preferred_element_type=jnp.float32) * (-0.5)
    qt_ref[...] = qt.reshape(1, _D, _BT)
    part = jnp.sum(m)

    i = pl.program_id(0)

    @pl.when(i == 0)
    def _init():
        acc_ref[0] = 0.0

    acc_ref[0] += part

    @pl.when(i == _GRID - 1)
    def _fini():
        loss_ref[0, 0] = acc_ref[0] * (1.25 / (_N * _D))


def _vq(x_cn, weight, xsq, wsq):
    return pl.pallas_call(
        _vq_body,
        grid=(_GRID,),
        in_specs=[
            pl.BlockSpec((_D, _BT), lambda i: (i, 0)),
            pl.BlockSpec((_K, _D), lambda i: (0, 0)),
            pl.BlockSpec((_BT, 1), lambda i: (i, 0)),
            pl.BlockSpec((1, _K), lambda i: (0, 0)),
        ],
        out_specs=[
            pl.BlockSpec((1, 1, _BT), lambda i: (i, 0, 0)),
            pl.BlockSpec(memory_space=pltpu.SMEM),
            pl.BlockSpec((1, _D, _BT), lambda i: (i, 0, 0)),
        ],
        out_shape=[
            jax.ShapeDtypeStruct((_GRID, 1, _BT), jnp.int32),
            jax.ShapeDtypeStruct((1, 1), jnp.float32),
            jax.ShapeDtypeStruct((_GRID, _D, _BT), jnp.float32),
        ],
        scratch_shapes=[pltpu.SMEM((1,), jnp.float32)],
    )(x_cn, weight, xsq, wsq)


def kernel(inputs, weight):
    x_cn = inputs.reshape(4 * _D, 32 * 32)
    xsq = jnp.sum(jnp.transpose(inputs, (0, 2, 3, 1)).reshape(-1, _D) ** 2,
                  axis=1, keepdims=True)
    wsq = jnp.sum(weight ** 2, axis=1).reshape(1, _K)
    idx3, loss, qt = _vq(x_cn, weight, xsq, wsq)
    quantized_st = qt.reshape(4, _D, 32, 32)
    return quantized_st, loss[0, 0], idx3.reshape(4, 32, 32)


# speedup vs baseline: 2.3270x; 1.2076x over previous
import jax
import jax.numpy as jnp
from jax import lax
from jax.experimental import pallas as pl
from jax.experimental.pallas import tpu as pltpu

_K = 1024
_D = 32
_N = 4096
_BT = 1024
_GRID = _N // _BT


def _vq_body(xt_ref, w_ref, xsq_ref, wsq_ref, idx_ref, loss_ref, qt_ref,
             acc_ref):
    xt = xt_ref[...]
    w2 = w_ref[...] * (-2.0)
    xsq = xsq_ref[...]
    wsq = wsq_ref[...]
    mm2 = lax.dot_general(w2, xt, (((1,), (0,)), ((), ())),
                          preferred_element_type=jnp.float32)
    d = (wsq + xsq) + mm2
    m = jnp.min(d, axis=0, keepdims=True)
    iota = lax.broadcasted_iota(jnp.int32, (_K, 1), 0).astype(jnp.float32)
    idxf = jnp.min(jnp.where(d == m, iota, jnp.float32(_K)), axis=0,
                   keepdims=True)
    idx_ref[...] = idxf.astype(jnp.int32).reshape(1, 1, _BT)
    onehot = jnp.where(iota == idxf, 1.0, 0.0)
    qt = lax.dot_general(w2, onehot, (((0,), (0,)), ((), ())),
                         preferred_element_type=jnp.float32) * (-0.5)
    qt_ref[...] = qt.reshape(1, _D, _BT)
    part = jnp.sum(m)

    i = pl.program_id(0)

    @pl.when(i == 0)
    def _init():
        acc_ref[0] = 0.0

    acc_ref[0] += part

    @pl.when(i == _GRID - 1)
    def _fini():
        loss_ref[0, 0] = acc_ref[0] * (1.25 / (_N * _D))


def _vq(x_cn, weight, xsq, wsq):
    return pl.pallas_call(
        _vq_body,
        grid=(_GRID,),
        in_specs=[
            pl.BlockSpec((_D, _BT), lambda i: (i, 0)),
            pl.BlockSpec((_K, _D), lambda i: (0, 0)),
            pl.BlockSpec((1, _BT), lambda i: (0, i)),
            pl.BlockSpec((_K, 1), lambda i: (0, 0)),
        ],
        out_specs=[
            pl.BlockSpec((1, 1, _BT), lambda i: (i, 0, 0)),
            pl.BlockSpec(memory_space=pltpu.SMEM),
            pl.BlockSpec((1, _D, _BT), lambda i: (i, 0, 0)),
        ],
        out_shape=[
            jax.ShapeDtypeStruct((_GRID, 1, _BT), jnp.int32),
            jax.ShapeDtypeStruct((1, 1), jnp.float32),
            jax.ShapeDtypeStruct((_GRID, _D, _BT), jnp.float32),
        ],
        scratch_shapes=[pltpu.SMEM((1,), jnp.float32)],
    )(x_cn, weight, xsq, wsq)


def kernel(inputs, weight):
    x_cn = inputs.reshape(4 * _D, 32 * 32)
    xsq = jnp.sum(jnp.transpose(inputs, (0, 2, 3, 1)).reshape(-1, _D) ** 2,
                  axis=1).reshape(1, _N)
    wsq = jnp.sum(weight ** 2, axis=1).reshape(_K, 1)
    idx3, loss, qt = _vq(x_cn, weight, xsq, wsq)
    quantized_st = qt.reshape(4, _D, 32, 32)
    return quantized_st, loss[0, 0], idx3.reshape(4, 32, 32)


# speedup vs baseline: 2.4373x; 1.0474x over previous
import jax
import jax.numpy as jnp
from jax import lax
from jax.experimental import pallas as pl
from jax.experimental.pallas import tpu as pltpu

_K = 1024
_D = 32
_N = 4096
_BT = 1024
_GRID = _N // _BT


def _vq_body(xt_ref, w_ref, wsq_ref, idx_ref, loss_ref, qt_ref, acc_ref):
    xt = xt_ref[...]
    w2 = w_ref[...] * (-2.0)
    wsq = wsq_ref[...]
    s = xt * xt
    xsq = s[0:1, :]
    for c in range(1, _D):
        xsq = xsq + s[c:c + 1, :]
    mm2 = lax.dot_general(w2, xt, (((1,), (0,)), ((), ())),
                          preferred_element_type=jnp.float32)
    d = (wsq + xsq) + mm2
    m = jnp.min(d, axis=0, keepdims=True)
    iota = lax.broadcasted_iota(jnp.int32, (_K, 1), 0).astype(jnp.float32)
    idxf = jnp.min(jnp.where(d == m, iota, jnp.float32(_K)), axis=0,
                   keepdims=True)
    idx_ref[...] = idxf.astype(jnp.int32).reshape(1, 1, _BT)
    onehot = jnp.where(iota == idxf, 1.0, 0.0)
    qt = lax.dot_general(w2, onehot, (((0,), (0,)), ((), ())),
                         preferred_element_type=jnp.float32) * (-0.5)
    qt_ref[...] = qt.reshape(1, _D, _BT)
    part = jnp.sum(m)

    i = pl.program_id(0)

    @pl.when(i == 0)
    def _init():
        acc_ref[0] = 0.0

    acc_ref[0] += part

    @pl.when(i == _GRID - 1)
    def _fini():
        loss_ref[0, 0] = acc_ref[0] * (1.25 / (_N * _D))


def _vq(x_cn, weight, wsq):
    return pl.pallas_call(
        _vq_body,
        grid=(_GRID,),
        in_specs=[
            pl.BlockSpec((_D, _BT), lambda i: (i, 0)),
            pl.BlockSpec((_K, _D), lambda i: (0, 0)),
            pl.BlockSpec((_K, 1), lambda i: (0, 0)),
        ],
        out_specs=[
            pl.BlockSpec((1, 1, _BT), lambda i: (i, 0, 0)),
            pl.BlockSpec(memory_space=pltpu.SMEM),
            pl.BlockSpec((1, _D, _BT), lambda i: (i, 0, 0)),
        ],
        out_shape=[
            jax.ShapeDtypeStruct((_GRID, 1, _BT), jnp.int32),
            jax.ShapeDtypeStruct((1, 1), jnp.float32),
            jax.ShapeDtypeStruct((_GRID, _D, _BT), jnp.float32),
        ],
        scratch_shapes=[pltpu.SMEM((1,), jnp.float32)],
    )(x_cn, weight, wsq)


def kernel(inputs, weight):
    x_cn = inputs.reshape(4 * _D, 32 * 32)
    wsq = jnp.sum(weight ** 2, axis=1).reshape(_K, 1)
    idx3, loss, qt = _vq(x_cn, weight, wsq)
    quantized_st = qt.reshape(4, _D, 32, 32)
    return quantized_st, loss[0, 0], idx3.reshape(4, 32, 32)
